# baseline (speedup 1.0000x reference)
import jax
import jax.numpy as jnp
from jax import lax
from jax.experimental import pallas as pl
from jax.experimental.pallas import tpu as pltpu
from jax.experimental.pallas import tpu_sc as plsc

B = 16384
S = 64
T = 64
NR2 = 50000
NW = 32
PER_W = B // NW
CHUNK = 64
NCHUNK = PER_W // CHUNK
NGROUP = CHUNK // 16

_C3 = -0.16666667
_C5 = 8.3333333e-3
_C7 = -1.9841270e-4
_C9 = 2.7557319e-6
_C11 = -2.5052108e-8


def _sin(x):
    x2 = x * x
    q = _C11
    q = q * x2 + _C9
    q = q * x2 + _C7
    q = q * x2 + _C5
    q = q * x2 + _C3
    return x * (1.0 + x2 * q)


def _sqrt(x):
    i = plsc.bitcast(x, jnp.int32)
    i = 0x5F3759DF - lax.shift_right_logical(i, 1)
    y = plsc.bitcast(i, jnp.float32)
    y = y * (1.5 - 0.5 * x * y * y)
    y = y * (1.5 - 0.5 * x * y * y)
    y = y * (1.5 - 0.5 * x * y * y)
    return x * y


def _body(r1_h, r2_h, r3_h, years_h, months_h, days_h, p2_h, p3_h,
          ret_h, re_h, yf_h, yp_h, ya_h, mf_h, mp_h, ma_h, df_h, dp_h, da_h,
          out_h,
          i1_v, i2_v, i3_v, i3h_v, po_v, yrs_v, mos_v, dys_v, p2_v, p3_v,
          out_v,
          r1r, r2r, r3r, yfr, ypr, yar, mfr, mpr, mar, dfr, dpr, dar,
          sem, sem2):
    wid = lax.axis_index("s") * 2 + lax.axis_index("c")
    base = wid * PER_W

    pltpu.sync_copy(r1_h.at[pl.ds(base, PER_W)], i1_v)
    pltpu.sync_copy(r2_h.at[pl.ds(base, PER_W)], i2_v)
    pltpu.sync_copy(r3_h.at[pl.ds(base, PER_W)], i3_v)
    pltpu.sync_copy(years_h.at[pl.ds(base, PER_W)], yrs_v)
    pltpu.sync_copy(months_h.at[pl.ds(base, PER_W)], mos_v)
    pltpu.sync_copy(days_h.at[pl.ds(base, PER_W)], dys_v)
    pltpu.sync_copy(p2_h, p2_v)
    pltpu.sync_copy(p3_h, p3_v)

    p2 = p2_v[...]
    p3 = p3_v[...]
    biota = lax.iota(jnp.int32, 16)
    zf = jnp.zeros((16,), jnp.float32)

    def prep(k, _):
        v = i3_v[pl.ds(k * 16, 16)]
        i3h_v[pl.ds(k * 16, 16)] = lax.shift_right_logical(v, 1)
        po_v[pl.ds(k * 16, 16)] = (v & 1) * S
        return 0

    lax.fori_loop(0, PER_W // 16, prep, 0)

    narrow = [(ret_h, r3r), (yf_h, yfr), (yp_h, ypr), (ya_h, yar),
              (mf_h, mfr), (mp_h, mpr), (ma_h, mar),
              (df_h, dfr), (dp_h, dpr), (da_h, dar)]

    for c in range(NCHUNK):
        o = c * CHUNK
        cps = [
            pltpu.async_copy(re_h.at[i1_v.at[pl.ds(o, CHUNK)]], r1r, sem),
            pltpu.async_copy(re_h.at[i2_v.at[pl.ds(o, CHUNK)]], r2r, sem),
        ] + [
            pltpu.async_copy(th.at[i3h_v.at[pl.ds(o, CHUNK)]], tb, sem)
            for th, tb in narrow
        ]
        for cp in cps:
            cp.wait()

        def group_step(g, _, o=o):
            yv16 = yrs_v[pl.ds(o + g * 16, 16)]
            mv16 = mos_v[pl.ds(o + g * 16, 16)]
            dv16 = dys_v[pl.ds(o + g * 16, 16)]

            pvec = po_v[pl.ds(o + g * 16, 16)]

            def estep(e, acc, g=g, yv16=yv16, mv16=mv16, dv16=dv16,
                      pvec=pvec):
                el = g * 16 + e
                efull = biota * 0 + e
                yt = jnp.take_along_axis(yv16, efull, axis=0)
                mt = jnp.take_along_axis(mv16, efull, axis=0)
                dt = jnp.take_along_axis(dv16, efull, axis=0)
                po = jnp.take_along_axis(pvec, efull, axis=0)[0]
                ss = zf
                for j in range(4):
                    a_lo = r1r[el, pl.ds(j * 16, 16)]
                    b_lo = r2r[el, pl.ds(j * 16, 16)]
                    c_lo = r3r[el, pl.ds(po + j * 16, 16)]
                    s_lo = a_lo + p2 * b_lo + p3 * c_lo
                    ss = ss + s_lo * s_lo

                    a_hi = r1r[el, pl.ds(S + j * 16, 16)]
                    b_hi = r2r[el, pl.ds(S + j * 16, 16)]
                    yfv = yfr[el, pl.ds(po + j * 16, 16)]
                    ypv = ypr[el, pl.ds(po + j * 16, 16)]
                    yav = yar[el, pl.ds(po + j * 16, 16)]
                    mfv = mfr[el, pl.ds(po + j * 16, 16)]
                    mpv = mpr[el, pl.ds(po + j * 16, 16)]
                    mav = mar[el, pl.ds(po + j * 16, 16)]
                    dfv = dfr[el, pl.ds(po + j * 16, 16)]
                    dpv = dpr[el, pl.ds(po + j * 16, 16)]
                    dav = dar[el, pl.ds(po + j * 16, 16)]
                    season = (yav * _sin(yfv * yt + ypv)
                              + mav * _sin(mfv * mt + mpv)
                              + dav * _sin(dfv * dt + dpv))
                    s_hi = a_hi + p2 * b_hi + p3 * season
                    ss = ss + s_hi * s_hi
                s = jnp.sum(ss)
                return jnp.where(biota == efull, zf + s, acc)

            acc = lax.fori_loop(0, 16, estep, zf)
            out_v[pl.ds(o + g * 16, 16)] = -_sqrt(acc)
            return 0

        lax.fori_loop(0, NGROUP, group_step, 0)

    pltpu.sync_copy(out_v, out_h.at[pl.ds(base, PER_W)])


@jax.jit
def _run(r1, r2, r3, years, months, days, p2b, p3b, ret, re,
         yf, yp, ya, mf, mp, ma, df, dp, da):
    mesh = plsc.VectorSubcoreMesh(core_axis_name="c", subcore_axis_name="s")
    f = pl.kernel(
        _body,
        out_type=jax.ShapeDtypeStruct((B,), jnp.float32),
        mesh=mesh,
        compiler_params=pltpu.CompilerParams(needs_layout_passes=False,
                                             use_tc_tiling_on_sc=True),
        scratch_types=[
            pltpu.VMEM((PER_W,), jnp.int32),
            pltpu.VMEM((PER_W,), jnp.int32),
            pltpu.VMEM((PER_W,), jnp.int32),
            pltpu.VMEM((PER_W,), jnp.int32),
            pltpu.VMEM((PER_W,), jnp.int32),
            pltpu.VMEM((PER_W,), jnp.float32),
            pltpu.VMEM((PER_W,), jnp.float32),
            pltpu.VMEM((PER_W,), jnp.float32),
            pltpu.VMEM((16,), jnp.float32),
            pltpu.VMEM((16,), jnp.float32),
            pltpu.VMEM((PER_W,), jnp.float32),
        ] + [pltpu.VMEM((CHUNK, S + T), jnp.float32)] * 12 + [
            pltpu.SemaphoreType.DMA,
            pltpu.SemaphoreType.DMA,
        ],
    )
    return f(r1, r2, r3, years, months, days, p2b, p3b, ret, re,
             yf, yp, ya, mf, mp, ma, df, dp, da)


def kernel(r1, r2, r3, years, months, days, p2, p3, rel_embs_t, rel_embs,
           y_freq, y_phi, y_amp, m_freq, m_phi, m_amp, d_freq, d_phi, d_amp):
    p2b = jnp.broadcast_to(p2.astype(jnp.float32), (16,))
    p3b = jnp.broadcast_to(p3.astype(jnp.float32), (16,))
    half = lambda t: t.reshape(NR2, 128)
    return _run(r1, r2, r3, years, months, days, p2b, p3b,
                half(rel_embs_t), rel_embs,
                half(y_freq), half(y_phi), half(y_amp),
                half(m_freq), half(m_phi), half(m_amp),
                half(d_freq), half(d_phi), half(d_amp))


# speedup vs baseline: 1.3529x; 1.3529x over previous
import jax
import jax.numpy as jnp
from jax import lax
from jax.experimental import pallas as pl
from jax.experimental.pallas import tpu as pltpu
from jax.experimental.pallas import tpu_sc as plsc

B = 16384
S = 64
T = 64
NW = 32
PER_W = B // NW
CHUNK = 64
NCHUNK = PER_W // CHUNK
NGROUP = CHUNK // 16

_C3 = -0.16666667
_C5 = 8.3333333e-3
_C7 = -1.9841270e-4
_C9 = 2.7557319e-6
_C11 = -2.5052108e-8


def _sin(x):
    x2 = x * x
    q = _C11
    q = q * x2 + _C9
    q = q * x2 + _C7
    q = q * x2 + _C5
    q = q * x2 + _C3
    return x * (1.0 + x2 * q)


def _sqrt(x):
    i = plsc.bitcast(x, jnp.int32)
    i = 0x5F3759DF - lax.shift_right_logical(i, 1)
    y = plsc.bitcast(i, jnp.float32)
    y = y * (1.5 - 0.5 * x * y * y)
    y = y * (1.5 - 0.5 * x * y * y)
    y = y * (1.5 - 0.5 * x * y * y)
    return x * y


def _body(r1_h, r2_h, r3_h, years_h, months_h, days_h, p2_h, p3_h,
          ret_h, re_h, yf_h, yp_h, ya_h, mf_h, mp_h, ma_h, df_h, dp_h, da_h,
          out_h,
          i1_v, i2_v, i3_v, yrs_v, mos_v, dys_v, p2_v, p3_v, out_v,
          r1r, r2r, r3r, yfr, ypr, yar, mfr, mpr, mar, dfr, dpr, dar,
          sem, sem2):
    wid = lax.axis_index("s") * 2 + lax.axis_index("c")
    base = wid * PER_W

    pltpu.sync_copy(r1_h.at[pl.ds(base, PER_W)], i1_v)
    pltpu.sync_copy(r2_h.at[pl.ds(base, PER_W)], i2_v)
    pltpu.sync_copy(r3_h.at[pl.ds(base, PER_W)], i3_v)
    pltpu.sync_copy(years_h.at[pl.ds(base, PER_W)], yrs_v)
    pltpu.sync_copy(months_h.at[pl.ds(base, PER_W)], mos_v)
    pltpu.sync_copy(days_h.at[pl.ds(base, PER_W)], dys_v)
    pltpu.sync_copy(p2_h, p2_v)
    pltpu.sync_copy(p3_h, p3_v)

    p2 = p2_v[...]
    p3 = p3_v[...]
    biota = lax.iota(jnp.int32, 16)
    zf = jnp.zeros((16,), jnp.float32)

    narrow = [(ret_h, r3r), (yf_h, yfr), (yp_h, ypr), (ya_h, yar),
              (mf_h, mfr), (mp_h, mpr), (ma_h, mar),
              (df_h, dfr), (dp_h, dpr), (da_h, dar)]

    for c in range(NCHUNK):
        o = c * CHUNK
        cps = [
            pltpu.async_copy(re_h.at[i1_v.at[pl.ds(o, CHUNK)]], r1r, sem),
            pltpu.async_copy(re_h.at[i2_v.at[pl.ds(o, CHUNK)]], r2r, sem),
        ]

        def fire_row(r, _, o=o):
            g16 = (r // 16) * 16
            vec = i3_v[pl.ds(o + g16, 16)]
            sel = jnp.take_along_axis(vec, biota * 0 + (r - g16), axis=0)
            ridx = sel[0]
            for th, tb in narrow:
                pltpu.async_copy(th.at[ridx], tb.at[r], sem2)
            return 0

        lax.fori_loop(0, CHUNK, fire_row, 0)
        for cp in cps:
            cp.wait()
        for th, tb in narrow:
            pltpu.make_async_copy(th.at[pl.ds(0, CHUNK)], tb, sem2).wait()

        def group_step(g, _, o=o):
            yv16 = yrs_v[pl.ds(o + g * 16, 16)]
            mv16 = mos_v[pl.ds(o + g * 16, 16)]
            dv16 = dys_v[pl.ds(o + g * 16, 16)]

            def estep(e, acc, g=g, yv16=yv16, mv16=mv16, dv16=dv16):
                el = g * 16 + e
                efull = biota * 0 + e
                yt = jnp.take_along_axis(yv16, efull, axis=0)
                mt = jnp.take_along_axis(mv16, efull, axis=0)
                dt = jnp.take_along_axis(dv16, efull, axis=0)
                ss = zf
                for j in range(4):
                    a_lo = r1r[el, pl.ds(j * 16, 16)]
                    b_lo = r2r[el, pl.ds(j * 16, 16)]
                    c_lo = r3r[el, pl.ds(j * 16, 16)]
                    s_lo = a_lo + p2 * b_lo + p3 * c_lo
                    ss = ss + s_lo * s_lo

                    a_hi = r1r[el, pl.ds(S + j * 16, 16)]
                    b_hi = r2r[el, pl.ds(S + j * 16, 16)]
                    yfv = yfr[el, pl.ds(j * 16, 16)]
                    ypv = ypr[el, pl.ds(j * 16, 16)]
                    yav = yar[el, pl.ds(j * 16, 16)]
                    mfv = mfr[el, pl.ds(j * 16, 16)]
                    mpv = mpr[el, pl.ds(j * 16, 16)]
                    mav = mar[el, pl.ds(j * 16, 16)]
                    dfv = dfr[el, pl.ds(j * 16, 16)]
                    dpv = dpr[el, pl.ds(j * 16, 16)]
                    dav = dar[el, pl.ds(j * 16, 16)]
                    season = (yav * _sin(yfv * yt + ypv)
                              + mav * _sin(mfv * mt + mpv)
                              + dav * _sin(dfv * dt + dpv))
                    s_hi = a_hi + p2 * b_hi + p3 * season
                    ss = ss + s_hi * s_hi
                s = jnp.sum(ss)
                return jnp.where(biota == efull, zf + s, acc)

            acc = lax.fori_loop(0, 16, estep, zf)
            out_v[pl.ds(o + g * 16, 16)] = -_sqrt(acc)
            return 0

        lax.fori_loop(0, NGROUP, group_step, 0)

    pltpu.sync_copy(out_v, out_h.at[pl.ds(base, PER_W)])


@jax.jit
def _run(r1, r2, r3, years, months, days, p2b, p3b, ret, re,
         yf, yp, ya, mf, mp, ma, df, dp, da):
    mesh = plsc.VectorSubcoreMesh(core_axis_name="c", subcore_axis_name="s")
    f = pl.kernel(
        _body,
        out_type=jax.ShapeDtypeStruct((B,), jnp.float32),
        mesh=mesh,
        compiler_params=pltpu.CompilerParams(needs_layout_passes=False,
                                             use_tc_tiling_on_sc=True),
        scratch_types=[
            pltpu.VMEM((PER_W,), jnp.int32),
            pltpu.VMEM((PER_W,), jnp.int32),
            pltpu.VMEM((PER_W,), jnp.int32),
            pltpu.VMEM((PER_W,), jnp.float32),
            pltpu.VMEM((PER_W,), jnp.float32),
            pltpu.VMEM((PER_W,), jnp.float32),
            pltpu.VMEM((16,), jnp.float32),
            pltpu.VMEM((16,), jnp.float32),
            pltpu.VMEM((PER_W,), jnp.float32),
            pltpu.VMEM((CHUNK, S + T), jnp.float32),
            pltpu.VMEM((CHUNK, S + T), jnp.float32),
            pltpu.VMEM((CHUNK, S), jnp.float32),
            pltpu.VMEM((CHUNK, T), jnp.float32),
            pltpu.VMEM((CHUNK, T), jnp.float32),
            pltpu.VMEM((CHUNK, T), jnp.float32),
            pltpu.VMEM((CHUNK, T), jnp.float32),
            pltpu.VMEM((CHUNK, T), jnp.float32),
            pltpu.VMEM((CHUNK, T), jnp.float32),
            pltpu.VMEM((CHUNK, T), jnp.float32),
            pltpu.VMEM((CHUNK, T), jnp.float32),
            pltpu.VMEM((CHUNK, T), jnp.float32),
            pltpu.SemaphoreType.DMA,
            pltpu.SemaphoreType.DMA,
        ],
    )
    return f(r1, r2, r3, years, months, days, p2b, p3b, ret, re,
             yf, yp, ya, mf, mp, ma, df, dp, da)


def kernel(r1, r2, r3, years, months, days, p2, p3, rel_embs_t, rel_embs,
           y_freq, y_phi, y_amp, m_freq, m_phi, m_amp, d_freq, d_phi, d_amp):
    p2b = jnp.broadcast_to(p2.astype(jnp.float32), (16,))
    p3b = jnp.broadcast_to(p3.astype(jnp.float32), (16,))
    return _run(r1, r2, r3, years, months, days, p2b, p3b,
                rel_embs_t, rel_embs,
                y_freq, y_phi, y_amp, m_freq, m_phi, m_amp,
                d_freq, d_phi, d_amp)


# speedup vs baseline: 1.4268x; 1.0546x over previous
import jax
import jax.numpy as jnp
from jax import lax
from jax.experimental import pallas as pl
from jax.experimental.pallas import tpu as pltpu
from jax.experimental.pallas import tpu_sc as plsc

B = 16384
S = 64
T = 64
NW = 32
PER_W = B // NW
CHUNK = 32
NCHUNK = PER_W // CHUNK
NGROUP = CHUNK // 16
NPAIR = NCHUNK // 2

_C3 = -0.16666667
_C5 = 8.3333333e-3
_C7 = -1.9841270e-4
_C9 = 2.7557319e-6
_C11 = -2.5052108e-8


def _sin(x):
    x2 = x * x
    q = _C11
    q = q * x2 + _C9
    q = q * x2 + _C7
    q = q * x2 + _C5
    q = q * x2 + _C3
    return x * (1.0 + x2 * q)


def _sqrt(x):
    i = plsc.bitcast(x, jnp.int32)
    i = 0x5F3759DF - lax.shift_right_logical(i, 1)
    y = plsc.bitcast(i, jnp.float32)
    y = y * (1.5 - 0.5 * x * y * y)
    y = y * (1.5 - 0.5 * x * y * y)
    y = y * (1.5 - 0.5 * x * y * y)
    return x * y


def _body(r1_h, r2_h, r3_h, years_h, months_h, days_h, p2_h, p3_h,
          ret_h, re_h, yf_h, yp_h, ya_h, mf_h, mp_h, ma_h, df_h, dp_h, da_h,
          out_h,
          i1_v, i2_v, i3_v, yrs_v, mos_v, dys_v, p2_v, p3_v, out_v,
          r1r, r2r, r3r, yfr, ypr, yar, mfr, mpr, mar, dfr, dpr, dar,
          r1rB, r2rB, r3rB, yfrB, yprB, yarB, mfrB, mprB, marB,
          dfrB, dprB, darB,
          sem, sem2, semB, sem2B):
    wid = lax.axis_index("s") * 2 + lax.axis_index("c")
    base = wid * PER_W

    pltpu.sync_copy(r1_h.at[pl.ds(base, PER_W)], i1_v)
    pltpu.sync_copy(r2_h.at[pl.ds(base, PER_W)], i2_v)
    pltpu.sync_copy(r3_h.at[pl.ds(base, PER_W)], i3_v)
    pltpu.sync_copy(years_h.at[pl.ds(base, PER_W)], yrs_v)
    pltpu.sync_copy(months_h.at[pl.ds(base, PER_W)], mos_v)
    pltpu.sync_copy(days_h.at[pl.ds(base, PER_W)], dys_v)
    pltpu.sync_copy(p2_h, p2_v)
    pltpu.sync_copy(p3_h, p3_v)

    p2 = p2_v[...]
    p3 = p3_v[...]
    biota = lax.iota(jnp.int32, 16)
    zf = jnp.zeros((16,), jnp.float32)

    sets = [
        dict(r1r=r1r, r2r=r2r, sem=sem, sem2=sem2,
             narrow=[(ret_h, r3r), (yf_h, yfr), (yp_h, ypr), (ya_h, yar),
                     (mf_h, mfr), (mp_h, mpr), (ma_h, mar),
                     (df_h, dfr), (dp_h, dpr), (da_h, dar)]),
        dict(r1r=r1rB, r2r=r2rB, sem=semB, sem2=sem2B,
             narrow=[(ret_h, r3rB), (yf_h, yfrB), (yp_h, yprB), (ya_h, yarB),
                     (mf_h, mfrB), (mp_h, mprB), (ma_h, marB),
                     (df_h, dfrB), (dp_h, dprB), (da_h, darB)]),
    ]

    def fire(c, st):
        o = c * CHUNK
        pltpu.async_copy(re_h.at[i1_v.at[pl.ds(o, CHUNK)]], st["r1r"],
                         st["sem"])
        pltpu.async_copy(re_h.at[i2_v.at[pl.ds(o, CHUNK)]], st["r2r"],
                         st["sem"])

        def fire_row(r, _):
            g16 = (r // 16) * 16
            vec = i3_v[pl.ds(o + g16, 16)]
            sel = jnp.take_along_axis(vec, biota * 0 + (r - g16), axis=0)
            ridx = sel[0]
            for th, tb in st["narrow"]:
                pltpu.async_copy(th.at[ridx], tb.at[r], st["sem2"])
            return 0

        lax.fori_loop(0, CHUNK, fire_row, 0)

    def drain(st):
        pltpu.make_async_copy(re_h.at[i1_v.at[pl.ds(0, CHUNK)]], st["r1r"],
                              st["sem"]).wait()
        pltpu.make_async_copy(re_h.at[i2_v.at[pl.ds(0, CHUNK)]], st["r2r"],
                              st["sem"]).wait()
        for th, tb in st["narrow"]:
            pltpu.make_async_copy(th.at[pl.ds(0, CHUNK)], tb,
                                  st["sem2"]).wait()

    def compute(o, st):
        r1r_c, r2r_c = st["r1r"], st["r2r"]
        (r3r_c, yfr_c, ypr_c, yar_c, mfr_c, mpr_c, mar_c,
         dfr_c, dpr_c, dar_c) = [tb for _, tb in st["narrow"]]

        def group_step(g, _, r1r=r1r_c, r2r=r2r_c, r3r=r3r_c,
                       yfr=yfr_c, ypr=ypr_c, yar=yar_c, mfr=mfr_c,
                       mpr=mpr_c, mar=mar_c, dfr=dfr_c, dpr=dpr_c,
                       dar=dar_c):
            yv16 = yrs_v[pl.ds(o + g * 16, 16)]
            mv16 = mos_v[pl.ds(o + g * 16, 16)]
            dv16 = dys_v[pl.ds(o + g * 16, 16)]

            def estep(e, acc, g=g, yv16=yv16, mv16=mv16, dv16=dv16):
                el = g * 16 + e
                efull = biota * 0 + e
                yt = jnp.take_along_axis(yv16, efull, axis=0)
                mt = jnp.take_along_axis(mv16, efull, axis=0)
                dt = jnp.take_along_axis(dv16, efull, axis=0)
                ss = zf
                for j in range(4):
                    a_lo = r1r[el, pl.ds(j * 16, 16)]
                    b_lo = r2r[el, pl.ds(j * 16, 16)]
                    c_lo = r3r[el, pl.ds(j * 16, 16)]
                    s_lo = a_lo + p2 * b_lo + p3 * c_lo
                    ss = ss + s_lo * s_lo

                    a_hi = r1r[el, pl.ds(S + j * 16, 16)]
                    b_hi = r2r[el, pl.ds(S + j * 16, 16)]
                    yfv = yfr[el, pl.ds(j * 16, 16)]
                    ypv = ypr[el, pl.ds(j * 16, 16)]
                    yav = yar[el, pl.ds(j * 16, 16)]
                    mfv = mfr[el, pl.ds(j * 16, 16)]
                    mpv = mpr[el, pl.ds(j * 16, 16)]
                    mav = mar[el, pl.ds(j * 16, 16)]
                    dfv = dfr[el, pl.ds(j * 16, 16)]
                    dpv = dpr[el, pl.ds(j * 16, 16)]
                    dav = dar[el, pl.ds(j * 16, 16)]
                    season = (yav * _sin(yfv * yt + ypv)
                              + mav * _sin(mfv * mt + mpv)
                              + dav * _sin(dfv * dt + dpv))
                    s_hi = a_hi + p2 * b_hi + p3 * season
                    ss = ss + s_hi * s_hi
                s = jnp.sum(ss)
                return jnp.where(biota == efull, zf + s, acc)

            acc = lax.fori_loop(0, 16, estep, zf)
            out_v[pl.ds(o + g * 16, 16)] = -_sqrt(acc)
            return 0

        lax.fori_loop(0, NGROUP, group_step, 0)

    fire(0, sets[0])

    def pair_step(p, _):
        c0 = 2 * p
        fire(c0 + 1, sets[1])
        drain(sets[0])
        compute(c0 * CHUNK, sets[0])

        @pl.when(p < NPAIR - 1)
        def _():
            fire(c0 + 2, sets[0])

        drain(sets[1])
        compute((c0 + 1) * CHUNK, sets[1])
        return 0

    lax.fori_loop(0, NPAIR, pair_step, 0)

    pltpu.sync_copy(out_v, out_h.at[pl.ds(base, PER_W)])


@jax.jit
def _run(r1, r2, r3, years, months, days, p2b, p3b, ret, re,
         yf, yp, ya, mf, mp, ma, df, dp, da):
    mesh = plsc.VectorSubcoreMesh(core_axis_name="c", subcore_axis_name="s")
    f = pl.kernel(
        _body,
        out_type=jax.ShapeDtypeStruct((B,), jnp.float32),
        mesh=mesh,
        compiler_params=pltpu.CompilerParams(needs_layout_passes=False,
                                             use_tc_tiling_on_sc=True),
        scratch_types=[
            pltpu.VMEM((PER_W,), jnp.int32),
            pltpu.VMEM((PER_W,), jnp.int32),
            pltpu.VMEM((PER_W,), jnp.int32),
            pltpu.VMEM((PER_W,), jnp.float32),
            pltpu.VMEM((PER_W,), jnp.float32),
            pltpu.VMEM((PER_W,), jnp.float32),
            pltpu.VMEM((16,), jnp.float32),
            pltpu.VMEM((16,), jnp.float32),
            pltpu.VMEM((PER_W,), jnp.float32),
            pltpu.VMEM((CHUNK, S + T), jnp.float32),
            pltpu.VMEM((CHUNK, S + T), jnp.float32),
            pltpu.VMEM((CHUNK, S), jnp.float32),
            pltpu.VMEM((CHUNK, T), jnp.float32),
            pltpu.VMEM((CHUNK, T), jnp.float32),
            pltpu.VMEM((CHUNK, T), jnp.float32),
            pltpu.VMEM((CHUNK, T), jnp.float32),
            pltpu.VMEM((CHUNK, T), jnp.float32),
            pltpu.VMEM((CHUNK, T), jnp.float32),
            pltpu.VMEM((CHUNK, T), jnp.float32),
            pltpu.VMEM((CHUNK, T), jnp.float32),
            pltpu.VMEM((CHUNK, T), jnp.float32),
            pltpu.VMEM((CHUNK, S + T), jnp.float32),
            pltpu.VMEM((CHUNK, S + T), jnp.float32),
            pltpu.VMEM((CHUNK, S), jnp.float32),
            pltpu.VMEM((CHUNK, T), jnp.float32),
            pltpu.VMEM((CHUNK, T), jnp.float32),
            pltpu.VMEM((CHUNK, T), jnp.float32),
            pltpu.VMEM((CHUNK, T), jnp.float32),
            pltpu.VMEM((CHUNK, T), jnp.float32),
            pltpu.VMEM((CHUNK, T), jnp.float32),
            pltpu.VMEM((CHUNK, T), jnp.float32),
            pltpu.VMEM((CHUNK, T), jnp.float32),
            pltpu.VMEM((CHUNK, T), jnp.float32),
            pltpu.SemaphoreType.DMA,
            pltpu.SemaphoreType.DMA,
            pltpu.SemaphoreType.DMA,
            pltpu.SemaphoreType.DMA,
        ],
    )
    return f(r1, r2, r3, years, months, days, p2b, p3b, ret, re,
             yf, yp, ya, mf, mp, ma, df, dp, da)


def kernel(r1, r2, r3, years, months, days, p2, p3, rel_embs_t, rel_embs,
           y_freq, y_phi, y_amp, m_freq, m_phi, m_amp, d_freq, d_phi, d_amp):
    p2b = jnp.broadcast_to(p2.astype(jnp.float32), (16,))
    p3b = jnp.broadcast_to(p3.astype(jnp.float32), (16,))
    return _run(r1, r2, r3, years, months, days, p2b, p3b,
                rel_embs_t, rel_embs,
                y_freq, y_phi, y_amp, m_freq, m_phi, m_amp,
                d_freq, d_phi, d_amp)


# speedup vs baseline: 1.4464x; 1.0138x over previous
import jax
import jax.numpy as jnp
from jax import lax
from jax.experimental import pallas as pl
from jax.experimental.pallas import tpu as pltpu
from jax.experimental.pallas import tpu_sc as plsc

B = 16384
S = 64
T = 64
NW = 32
PER_W = B // NW
CHUNK_A = 64
NCHUNK_A = PER_W // CHUNK_A
CHUNK = 32
NCHUNK = PER_W // CHUNK
NGROUP = CHUNK // 16
NPAIR = NCHUNK // 2

_C3 = -0.16666667
_C5 = 8.3333333e-3
_C7 = -1.9841270e-4
_C9 = 2.7557319e-6


def _sin(x):
    x2 = x * x
    q = _C9
    q = q * x2 + _C7
    q = q * x2 + _C5
    q = q * x2 + _C3
    return x * (1.0 + x2 * q)


def _sqrt(x):
    i = plsc.bitcast(x, jnp.int32)
    i = 0x5F3759DF - lax.shift_right_logical(i, 1)
    y = plsc.bitcast(i, jnp.float32)
    y = y * (1.5 - 0.5 * x * y * y)
    y = y * (1.5 - 0.5 * x * y * y)
    y = y * (1.5 - 0.5 * x * y * y)
    return x * y


def _body_a(r1_h, r2_h, p2_h, re_h, ab_h,
            i1_v, i2_v, p2_v, r1r, r2r, abv, sem):
    wid = lax.axis_index("s") * 2 + lax.axis_index("c")
    base = wid * PER_W

    pltpu.sync_copy(r1_h.at[pl.ds(base, PER_W)], i1_v)
    pltpu.sync_copy(r2_h.at[pl.ds(base, PER_W)], i2_v)
    pltpu.sync_copy(p2_h, p2_v)
    p2 = p2_v[...]

    for c in range(NCHUNK_A):
        o = c * CHUNK_A
        cp1 = pltpu.async_copy(re_h.at[i1_v.at[pl.ds(o, CHUNK_A)]], r1r, sem)
        cp2 = pltpu.async_copy(re_h.at[i2_v.at[pl.ds(o, CHUNK_A)]], r2r, sem)
        cp1.wait()
        cp2.wait()

        def row_step(r, _):
            for j in range(8):
                abv[r, pl.ds(j * 16, 16)] = (
                    r1r[r, pl.ds(j * 16, 16)]
                    + p2 * r2r[r, pl.ds(j * 16, 16)])
            return 0

        lax.fori_loop(0, CHUNK_A, row_step, 0)
        pltpu.sync_copy(abv, ab_h.at[pl.ds(base + o, CHUNK_A), :])


def _body_b(r3_h, years_h, months_h, days_h, p3_h, ab_h,
            ret_h, yf_h, yp_h, ya_h, mf_h, mp_h, ma_h, df_h, dp_h, da_h,
            out_h,
            i3_v, yrs_v, mos_v, dys_v, p3_v, out_v,
            abr, r3r, yfr, ypr, yar, mfr, mpr, mar, dfr, dpr, dar,
            abrB, r3rB, yfrB, yprB, yarB, mfrB, mprB, marB, dfrB, dprB, darB,
            sem, sem2, semB, sem2B):
    wid = lax.axis_index("s") * 2 + lax.axis_index("c")
    base = wid * PER_W

    pltpu.sync_copy(r3_h.at[pl.ds(base, PER_W)], i3_v)
    pltpu.sync_copy(years_h.at[pl.ds(base, PER_W)], yrs_v)
    pltpu.sync_copy(months_h.at[pl.ds(base, PER_W)], mos_v)
    pltpu.sync_copy(days_h.at[pl.ds(base, PER_W)], dys_v)
    pltpu.sync_copy(p3_h, p3_v)

    p3 = p3_v[...]
    biota = lax.iota(jnp.int32, 16)
    zf = jnp.zeros((16,), jnp.float32)

    sets = [
        dict(abr=abr, sem=sem, sem2=sem2,
             narrow=[(ret_h, r3r), (yf_h, yfr), (yp_h, ypr), (ya_h, yar),
                     (mf_h, mfr), (mp_h, mpr), (ma_h, mar),
                     (df_h, dfr), (dp_h, dpr), (da_h, dar)]),
        dict(abr=abrB, sem=semB, sem2=sem2B,
             narrow=[(ret_h, r3rB), (yf_h, yfrB), (yp_h, yprB), (ya_h, yarB),
                     (mf_h, mfrB), (mp_h, mprB), (ma_h, marB),
                     (df_h, dfrB), (dp_h, dprB), (da_h, darB)]),
    ]

    def fire(c, st):
        o = c * CHUNK
        pltpu.async_copy(ab_h.at[pl.ds(base + o, CHUNK), :], st["abr"],
                         st["sem"])

        def fire_row(r, _):
            g16 = (r // 16) * 16
            vec = i3_v[pl.ds(o + g16, 16)]
            sel = jnp.take_along_axis(vec, biota * 0 + (r - g16), axis=0)
            ridx = sel[0]
            for th, tb in st["narrow"]:
                pltpu.async_copy(th.at[ridx], tb.at[r], st["sem2"])
            return 0

        lax.fori_loop(0, CHUNK, fire_row, 0)

    def drain(st):
        pltpu.make_async_copy(ab_h.at[pl.ds(0, CHUNK), :], st["abr"],
                              st["sem"]).wait()
        for th, tb in st["narrow"]:
            pltpu.make_async_copy(th.at[pl.ds(0, CHUNK)], tb,
                                  st["sem2"]).wait()

    def compute(o, st):
        abr_c = st["abr"]
        (r3r_c, yfr_c, ypr_c, yar_c, mfr_c, mpr_c, mar_c,
         dfr_c, dpr_c, dar_c) = [tb for _, tb in st["narrow"]]

        def group_step(g, _, abr=abr_c, r3r=r3r_c,
                       yfr=yfr_c, ypr=ypr_c, yar=yar_c, mfr=mfr_c,
                       mpr=mpr_c, mar=mar_c, dfr=dfr_c, dpr=dpr_c,
                       dar=dar_c):
            yv16 = yrs_v[pl.ds(o + g * 16, 16)]
            mv16 = mos_v[pl.ds(o + g * 16, 16)]
            dv16 = dys_v[pl.ds(o + g * 16, 16)]

            def estep(e, acc, g=g, yv16=yv16, mv16=mv16, dv16=dv16):
                el = g * 16 + e
                efull = biota * 0 + e
                yt = jnp.take_along_axis(yv16, efull, axis=0)
                mt = jnp.take_along_axis(mv16, efull, axis=0)
                dt = jnp.take_along_axis(dv16, efull, axis=0)
                ss = zf
                for j in range(4):
                    a_lo = abr[el, pl.ds(j * 16, 16)]
                    c_lo = r3r[el, pl.ds(j * 16, 16)]
                    s_lo = a_lo + p3 * c_lo
                    ss = ss + s_lo * s_lo

                    a_hi = abr[el, pl.ds(S + j * 16, 16)]
                    yfv = yfr[el, pl.ds(j * 16, 16)]
                    ypv = ypr[el, pl.ds(j * 16, 16)]
                    yav = yar[el, pl.ds(j * 16, 16)]
                    mfv = mfr[el, pl.ds(j * 16, 16)]
                    mpv = mpr[el, pl.ds(j * 16, 16)]
                    mav = mar[el, pl.ds(j * 16, 16)]
                    dfv = dfr[el, pl.ds(j * 16, 16)]
                    dpv = dpr[el, pl.ds(j * 16, 16)]
                    dav = dar[el, pl.ds(j * 16, 16)]
                    season = (yav * _sin(yfv * yt + ypv)
                              + mav * _sin(mfv * mt + mpv)
                              + dav * _sin(dfv * dt + dpv))
                    s_hi = a_hi + p3 * season
                    ss = ss + s_hi * s_hi
                s = jnp.sum(ss)
                return jnp.where(biota == efull, zf + s, acc)

            acc = lax.fori_loop(0, 16, estep, zf)
            out_v[pl.ds(o + g * 16, 16)] = -_sqrt(acc)
            return 0

        lax.fori_loop(0, NGROUP, group_step, 0)

    fire(0, sets[0])

    def pair_step(p, _):
        c0 = 2 * p
        fire(c0 + 1, sets[1])
        drain(sets[0])
        compute(c0 * CHUNK, sets[0])

        @pl.when(p < NPAIR - 1)
        def _():
            fire(c0 + 2, sets[0])

        drain(sets[1])
        compute((c0 + 1) * CHUNK, sets[1])
        return 0

    lax.fori_loop(0, NPAIR, pair_step, 0)

    pltpu.sync_copy(out_v, out_h.at[pl.ds(base, PER_W)])


_PARAMS = pltpu.CompilerParams(needs_layout_passes=False,
                               use_tc_tiling_on_sc=True)


@jax.jit
def _run(r1, r2, r3, years, months, days, p2b, p3b, ret, re,
         yf, yp, ya, mf, mp, ma, df, dp, da):
    mesh = plsc.VectorSubcoreMesh(core_axis_name="c", subcore_axis_name="s")
    fa = pl.kernel(
        _body_a,
        out_type=jax.ShapeDtypeStruct((B, S + T), jnp.float32),
        mesh=mesh,
        compiler_params=_PARAMS,
        scratch_types=[
            pltpu.VMEM((PER_W,), jnp.int32),
            pltpu.VMEM((PER_W,), jnp.int32),
            pltpu.VMEM((16,), jnp.float32),
            pltpu.VMEM((CHUNK_A, S + T), jnp.float32),
            pltpu.VMEM((CHUNK_A, S + T), jnp.float32),
            pltpu.VMEM((CHUNK_A, S + T), jnp.float32),
            pltpu.SemaphoreType.DMA,
        ],
    )
    ab = fa(r1, r2, p2b, re)

    fb = pl.kernel(
        _body_b,
        out_type=jax.ShapeDtypeStruct((B,), jnp.float32),
        mesh=mesh,
        compiler_params=_PARAMS,
        scratch_types=[
            pltpu.VMEM((PER_W,), jnp.int32),
            pltpu.VMEM((PER_W,), jnp.float32),
            pltpu.VMEM((PER_W,), jnp.float32),
            pltpu.VMEM((PER_W,), jnp.float32),
            pltpu.VMEM((16,), jnp.float32),
            pltpu.VMEM((PER_W,), jnp.float32),
        ] + [
            pltpu.VMEM((CHUNK, S + T), jnp.float32),
            pltpu.VMEM((CHUNK, S), jnp.float32),
        ] + [pltpu.VMEM((CHUNK, T), jnp.float32)] * 9 + [
            pltpu.VMEM((CHUNK, S + T), jnp.float32),
            pltpu.VMEM((CHUNK, S), jnp.float32),
        ] + [pltpu.VMEM((CHUNK, T), jnp.float32)] * 9 + [
            pltpu.SemaphoreType.DMA,
            pltpu.SemaphoreType.DMA,
            pltpu.SemaphoreType.DMA,
            pltpu.SemaphoreType.DMA,
        ],
    )
    return fb(r3, years, months, days, p3b, ab,
              ret, yf, yp, ya, mf, mp, ma, df, dp, da)


def kernel(r1, r2, r3, years, months, days, p2, p3, rel_embs_t, rel_embs,
           y_freq, y_phi, y_amp, m_freq, m_phi, m_amp, d_freq, d_phi, d_amp):
    p2b = jnp.broadcast_to(p2.astype(jnp.float32), (16,))
    p3b = jnp.broadcast_to(p3.astype(jnp.float32), (16,))
    return _run(r1, r2, r3, years, months, days, p2b, p3b,
                rel_embs_t, rel_embs,
                y_freq, y_phi, y_amp, m_freq, m_phi, m_amp,
                d_freq, d_phi, d_amp)
